# baseline (speedup 1.0000x reference)
import functools

import jax
import jax.numpy as jnp
from jax import lax
from jax.experimental import pallas as pl
from jax.experimental.pallas import tpu as pltpu
from jax.experimental.pallas import tpu_sc as plsc

_L = 16


@functools.lru_cache(maxsize=None)
def _make_sc_kernel(B: int, NU: int, NI: int, K: int):
    info = plsc.get_sparse_core_info()
    NC, NS = info.num_cores, info.num_subcores
    NW = NC * NS
    assert B % (8 * NW) == 0
    b_per_w = B // NW
    chunk = 128
    assert b_per_w % chunk == 0
    n_chunks = b_per_w // chunk
    assert K == _L

    mesh = plsc.VectorSubcoreMesh(core_axis_name="c", subcore_axis_name="s")

    @functools.partial(
        pl.kernel,
        mesh=mesh,
        compiler_params=pltpu.CompilerParams(
            needs_layout_passes=False, use_tc_tiling_on_sc=False),
        out_type=jax.ShapeDtypeStruct((B,), jnp.float32),
        scratch_types=[
            pltpu.VMEM((n_chunks, chunk), jnp.int32),
            pltpu.VMEM((n_chunks, chunk), jnp.int32),
            pltpu.VMEM((b_per_w, K), jnp.float32),
            pltpu.VMEM((b_per_w, K), jnp.float32),
            pltpu.VMEM((b_per_w,), jnp.float32),
            pltpu.SemaphoreType.DMA,
        ],
    )
    def mf_kernel(uidx_hbm, iidx_hbm, w_hbm, h_hbm, out_hbm,
                  uidx_v, iidx_v, urows, vrows, outv, sem):
        wid = lax.axis_index("s") * NC + lax.axis_index("c")
        base = wid * b_per_w

        for j in range(n_chunks):
            pltpu.sync_copy(uidx_hbm.at[pl.ds(base + j * chunk, chunk)],
                            uidx_v.at[j])
            pltpu.sync_copy(iidx_hbm.at[pl.ds(base + j * chunk, chunk)],
                            iidx_v.at[j])

        copies = []
        for j in range(n_chunks):
            copies.append(pltpu.async_copy(
                w_hbm.at[uidx_v.at[j]],
                urows.at[pl.ds(j * chunk, chunk)], sem))
            copies.append(pltpu.async_copy(
                h_hbm.at[iidx_v.at[j]],
                vrows.at[pl.ds(j * chunk, chunk)], sem))
        for c in copies:
            c.wait()

        lanes = lax.iota(jnp.int32, _L)

        def body(g, carry):
            rows = g * _L + lanes
            acc = jnp.zeros((_L,), jnp.float32)
            for d in range(K):
                cols = jnp.full((_L,), d, jnp.int32)
                u = plsc.load_gather(urows, [rows, cols])
                v = plsc.load_gather(vrows, [rows, cols])
                acc = acc + u * v
            outv[pl.ds(g * _L, _L)] = 1.0 / (1.0 + jnp.exp(-acc))
            return carry

        lax.fori_loop(0, b_per_w // _L, body, 0)

        pltpu.sync_copy(outv, out_hbm.at[pl.ds(base, b_per_w)])

    return mf_kernel


def kernel(x, W, H):
    user_idx = x[:, 0].astype(jnp.int32)
    item_idx = x[:, 1].astype(jnp.int32)
    B = x.shape[0]
    fn = _make_sc_kernel(B, W.shape[0], H.shape[0], W.shape[1])
    return fn(user_idx, item_idx, W, H)
